# baseline (speedup 1.0000x reference)
import jax
import jax.numpy as jnp
from jax.experimental import pallas as pl
from jax.experimental.pallas import tpu as pltpu


def _make_gcn_kernel(bm, half):
    def _gcn_kernel(adj0_ref, adj1_ref, x_ref, w1_ref, b1_ref, w2_ref,
                    b2_ref, out_ref, s1_ref, h2_ref):
        p = pl.program_id(0)
        m = pl.program_id(1)

        @pl.when((p == 0) & (m == 0))
        def _():
            s1_ref[...] = jnp.dot(
                x_ref[...], w1_ref[...],
                preferred_element_type=jnp.float32)

        @pl.when(p == 0)
        def _():
            for i, a_ref in enumerate((adj0_ref, adj1_ref)):
                acc = jnp.dot(a_ref[...], s1_ref[...],
                              preferred_element_type=jnp.float32)
                h = jnp.maximum(acc + b1_ref[...], 0.0)
                h2 = jnp.dot(h, w2_ref[...],
                             preferred_element_type=jnp.float32)
                h2_ref[pl.ds(m * bm + i * half, half), :] = h2
                out_ref[0, i * half:(i + 1) * half, :] = h2

        @pl.when(p == 1)
        def _():
            for i, a_ref in enumerate((adj0_ref, adj1_ref)):
                acc = jnp.dot(a_ref[...], h2_ref[...],
                              preferred_element_type=jnp.float32)
                out_ref[0, i * half:(i + 1) * half, :] = acc + b2_ref[...]

    return _gcn_kernel


def _pick_bm(n):
    for bm in (400, 200, 80, 40, 8):
        if n % bm == 0:
            return bm
    return min(n, 256)


def _gcn(adj, x, w1, b1, w2, b2):
    n, f = x.shape
    bm = _pick_bm(n)
    full = lambda r, c: pl.BlockSpec((r, c), lambda p, m: (0, 0))

    half = bm // 2
    return pl.pallas_call(
        _make_gcn_kernel(bm, half),
        grid=(2, pl.cdiv(n, bm)),
        in_specs=[
            pl.BlockSpec((half, n), lambda p, m: (2 * m, 0)),
            pl.BlockSpec((half, n), lambda p, m: (2 * m + 1, 0)),
            full(n, f),
            full(f, f),
            full(1, f),
            full(f, f),
            full(1, f),
        ],
        out_specs=pl.BlockSpec((1, bm, f), lambda p, m: (p, m, 0)),
        out_shape=jax.ShapeDtypeStruct((2, n, f), jnp.float32),
        scratch_shapes=[
            pltpu.VMEM((n, f), jnp.float32),
            pltpu.VMEM((n, f), jnp.float32),
        ],
        compiler_params=pltpu.CompilerParams(
            dimension_semantics=("arbitrary", "arbitrary")),
    )(adj, adj, x, w1, b1.reshape(1, f), w2, b2.reshape(1, f))[1]


def kernel(drug_graph, drug_sim_feat, dis_graph, disease_sim_feat,
           W1_drug, b1_drug, W2_drug, b2_drug,
           W1_dis, b1_dis, W2_dis, b2_dis):
    emb1 = _gcn(drug_graph, drug_sim_feat, W1_drug, b1_drug, W2_drug, b2_drug)
    emb2 = _gcn(dis_graph, disease_sim_feat, W1_dis, b1_dis, W2_dis, b2_dis)
    return (emb1, emb2, emb1, emb2)


# speedup vs baseline: 1.0606x; 1.0606x over previous
import jax
import jax.numpy as jnp
from jax.experimental import pallas as pl
from jax.experimental.pallas import tpu as pltpu


def _make_gcn_kernel(bm):
    def _gcn_kernel(adj_ref, x_ref, w1_ref, b1_ref, w2_ref, b2_ref,
                    out_ref, s1_ref, h2_ref):
        p = pl.program_id(0)
        m = pl.program_id(1)

        @pl.when((p == 0) & (m == 0))
        def _():
            s1_ref[...] = jnp.dot(
                x_ref[...], w1_ref[...],
                preferred_element_type=jnp.float32)

        @pl.when(p == 0)
        def _():
            acc = jnp.dot(adj_ref[...], s1_ref[...],
                          preferred_element_type=jnp.float32)
            h = jnp.maximum(acc + b1_ref[...], 0.0)
            h2_ref[pl.ds(m * bm, bm), :] = jnp.dot(
                h, w2_ref[...], preferred_element_type=jnp.float32)

        @pl.when(p == 1)
        def _():
            acc = jnp.dot(adj_ref[...], h2_ref[...],
                          preferred_element_type=jnp.float32)
            out_ref[...] = acc + b2_ref[...]

    return _gcn_kernel


def _pick_bm(n):
    for bm in (400, 200, 80, 40, 8):
        if n % bm == 0:
            return bm
    return min(n, 256)


def _gcn(adj, x, w1, b1, w2, b2):
    n, f = x.shape
    bm = _pick_bm(n)
    full = lambda r, c: pl.BlockSpec((r, c), lambda p, m: (0, 0))

    return pl.pallas_call(
        _make_gcn_kernel(bm),
        grid=(2, pl.cdiv(n, bm)),
        in_specs=[
            pl.BlockSpec((bm, n), lambda p, m: (m, 0)),
            full(n, f),
            full(f, f),
            full(1, f),
            full(f, f),
            full(1, f),
        ],
        out_specs=pl.BlockSpec((bm, f), lambda p, m: (m * p, 0)),
        out_shape=jax.ShapeDtypeStruct((n, f), jnp.float32),
        scratch_shapes=[
            pltpu.VMEM((n, f), jnp.float32),
            pltpu.VMEM((n, f), jnp.float32),
        ],
        compiler_params=pltpu.CompilerParams(
            dimension_semantics=("arbitrary", "arbitrary")),
    )(adj, x, w1, b1.reshape(1, f), w2, b2.reshape(1, f))


def kernel(drug_graph, drug_sim_feat, dis_graph, disease_sim_feat,
           W1_drug, b1_drug, W2_drug, b2_drug,
           W1_dis, b1_dis, W2_dis, b2_dis):
    emb1 = _gcn(drug_graph, drug_sim_feat, W1_drug, b1_drug, W2_drug, b2_drug)
    emb2 = _gcn(dis_graph, disease_sim_feat, W1_dis, b1_dis, W2_dis, b2_dis)
    return (emb1, emb2, emb1, emb2)
